# baseline (speedup 1.0000x reference)
import functools

import jax
import jax.numpy as jnp
from jax import lax
from jax.experimental import pallas as pl
from jax.experimental.pallas import tpu as pltpu
from jax.experimental.pallas import tpu_sc as plsc

B = 16384
F = 100
D = 16
V = 101
L = 16
XPAD = 128
XSTRIDE = 113
NC = 2
NS = 16
NW = NC * NS
ROWS_PER_W = B // NW
GROUPS = ROWS_PER_W // L


def _fm_body(x_hbm, embP_hbm, nrm_hbm, wR_hbm, b_hbm, out_hbm,
             x_r, embP_v, nrm_v, wR_v, b_v, out_v):
    wid = lax.axis_index("s") * NC + lax.axis_index("c")
    row0 = wid * ROWS_PER_W

    pltpu.sync_copy(x_hbm.at[pl.ds(row0, ROWS_PER_W), :], x_r)
    pltpu.sync_copy(embP_hbm, embP_v)
    pltpu.sync_copy(nrm_hbm, nrm_v)
    pltpu.sync_copy(wR_hbm, wR_v)
    pltpu.sync_copy(b_hbm, b_v)

    lanes = lax.iota(jnp.int32, L)
    zeros = jnp.zeros((L,), jnp.float32)
    hi_mask = jnp.full((L,), -65536, jnp.int32)

    def g_body(g, carry):
        rows = lanes + g * L

        def f_body(f, fcarry):
            wf, lin, S, q = fcarry
            xv = plsc.load_gather(x_r, [rows, wf])
            xc = jnp.clip(xv, 0.0, float(V - 1))
            idx = xc.astype(jnp.int32)
            wv = plsc.load_gather(wR_v, [wf * L + lanes])
            lin = lin + wv * xc
            ea = idx * L + lanes
            nv = plsc.load_gather(nrm_v, [ea])
            q = q + nv
            S_new = []
            for p in range(D // 2):
                w2 = plsc.load_gather(embP_v.at[pl.ds(p * V * L, V * L)], [ea])
                e0 = lax.bitcast_convert_type(
                    jnp.left_shift(w2, 16), jnp.float32)
                e1 = lax.bitcast_convert_type(
                    jnp.bitwise_and(w2, hi_mask), jnp.float32)
                S_new.append(S[2 * p] + e0)
                S_new.append(S[2 * p + 1] + e1)
            wf = wf + 1
            wf = jnp.where(wf >= F, wf - F, wf)
            return wf, lin, tuple(S_new), q

        init = (lanes, b_v[...], (zeros,) * D, zeros)
        _, lin, S, q = lax.fori_loop(0, F, f_body, init, unroll=4)
        sq = zeros
        for d in range(D):
            sq = sq + S[d] * S[d]
        out_v[pl.ds(g * L, L)] = lin + 0.5 * (sq - q)
        return carry

    lax.fori_loop(0, GROUPS, g_body, 0)
    pltpu.sync_copy(out_v, out_hbm.at[pl.ds(row0, ROWS_PER_W)])


def _make_sc_call(interpret=False):
    mesh = plsc.VectorSubcoreMesh(core_axis_name="c", subcore_axis_name="s")
    return pl.kernel(
        _fm_body,
        out_type=jax.ShapeDtypeStruct((B,), jnp.float32),
        mesh=mesh,
        scratch_types=[
            pltpu.VMEM((ROWS_PER_W, XPAD), jnp.float32),
            pltpu.VMEM((D // 2 * V * L,), jnp.int32),
            pltpu.VMEM((V * L,), jnp.float32),
            pltpu.VMEM((F * L,), jnp.float32),
            pltpu.VMEM((L,), jnp.float32),
            pltpu.VMEM((ROWS_PER_W,), jnp.float32),
        ],
        compiler_params=pltpu.CompilerParams(
            use_tc_tiling_on_sc=False,
            needs_layout_passes=False,
        ),
        interpret=interpret,
    )


@jax.jit
def kernel(x, W_lin, b_lin, emb):
    xf = jnp.pad(x, ((0, 0), (0, XPAD - F)))
    u = lax.bitcast_convert_type(
        emb.astype(jnp.bfloat16), jnp.uint16).astype(jnp.uint32)
    pair = u[:, 0::2] | (u[:, 1::2] << 16)
    embP = jnp.repeat(lax.bitcast_convert_type(pair, jnp.int32).T,
                      L, axis=1).reshape(-1)
    nrm = jnp.repeat(jnp.sum(emb * emb, axis=1), L)
    wR = jnp.repeat(W_lin[0], L)
    bf = jnp.full((L,), b_lin[0], jnp.float32)
    return _make_sc_call()(xf, embP, nrm, wR, bf)


# speedup vs baseline: 1.0512x; 1.0512x over previous
import functools

import jax
import jax.numpy as jnp
from jax import lax
from jax.experimental import pallas as pl
from jax.experimental.pallas import tpu as pltpu
from jax.experimental.pallas import tpu_sc as plsc

B = 16384
F = 100
D = 16
V = 101
L = 16
XPAD = 128
XSTRIDE = 113
NC = 2
NS = 16
NW = NC * NS
ROWS_PER_W = B // NW
GROUPS = ROWS_PER_W // L


def _fm_body(x_hbm, embP_hbm, nrm_hbm, wR_hbm, b_hbm, out_hbm,
             x_r, embP_v, nrm_v, wR_v, b_v, out_v):
    wid = lax.axis_index("s") * NC + lax.axis_index("c")
    row0 = wid * ROWS_PER_W

    pltpu.sync_copy(x_hbm.at[pl.ds(row0 * XPAD, ROWS_PER_W * XPAD)], x_r)
    pltpu.sync_copy(embP_hbm, embP_v)
    pltpu.sync_copy(nrm_hbm, nrm_v)
    pltpu.sync_copy(wR_hbm, wR_v)
    pltpu.sync_copy(b_hbm, b_v)

    lanes = lax.iota(jnp.int32, L)
    zeros = jnp.zeros((L,), jnp.float32)
    hi_mask = jnp.full((L,), -65536, jnp.int32)

    def g_body(g, carry):
        rows = lanes + g * L
        rbase = rows * XPAD

        def f_body(f, fcarry):
            wf, lin, S, q = fcarry
            xv = plsc.load_gather(x_r, [rbase + wf])
            xc = jnp.clip(xv, 0.0, float(V - 1))
            idx = xc.astype(jnp.int32)
            wv = plsc.load_gather(wR_v, [wf * L + lanes])
            lin = lin + wv * xc
            ea = idx * L + lanes
            nv = plsc.load_gather(nrm_v, [ea])
            q = q + nv
            S_new = []
            for p in range(D // 2):
                w2 = plsc.load_gather(embP_v.at[pl.ds(p * V * L, V * L)], [ea])
                e0 = lax.bitcast_convert_type(
                    jnp.left_shift(w2, 16), jnp.float32)
                e1 = lax.bitcast_convert_type(
                    jnp.bitwise_and(w2, hi_mask), jnp.float32)
                S_new.append(S[2 * p] + e0)
                S_new.append(S[2 * p + 1] + e1)
            wf = wf + 1
            wf = jnp.where(wf >= F, wf - F, wf)
            return wf, lin, tuple(S_new), q

        init = (lanes, b_v[...], (zeros,) * D, zeros)
        _, lin, S, q = lax.fori_loop(0, F, f_body, init, unroll=4)
        sq = zeros
        for d in range(D):
            sq = sq + S[d] * S[d]
        out_v[pl.ds(g * L, L)] = lin + 0.5 * (sq - q)
        return carry

    lax.fori_loop(0, GROUPS, g_body, 0)
    pltpu.sync_copy(out_v, out_hbm.at[pl.ds(row0, ROWS_PER_W)])


def _make_sc_call(interpret=False):
    mesh = plsc.VectorSubcoreMesh(core_axis_name="c", subcore_axis_name="s")
    return pl.kernel(
        _fm_body,
        out_type=jax.ShapeDtypeStruct((B,), jnp.float32),
        mesh=mesh,
        scratch_types=[
            pltpu.VMEM((ROWS_PER_W * XPAD,), jnp.float32),
            pltpu.VMEM((D // 2 * V * L,), jnp.int32),
            pltpu.VMEM((V * L,), jnp.float32),
            pltpu.VMEM((F * L,), jnp.float32),
            pltpu.VMEM((L,), jnp.float32),
            pltpu.VMEM((ROWS_PER_W,), jnp.float32),
        ],
        compiler_params=pltpu.CompilerParams(
            use_tc_tiling_on_sc=False,
            needs_layout_passes=False,
        ),
        interpret=interpret,
    )


@jax.jit
def kernel(x, W_lin, b_lin, emb):
    xf = jnp.pad(x, ((0, 0), (0, XPAD - F))).reshape(-1)
    u = lax.bitcast_convert_type(
        emb.astype(jnp.bfloat16), jnp.uint16).astype(jnp.uint32)
    pair = u[:, 0::2] | (u[:, 1::2] << 16)
    embP = jnp.repeat(lax.bitcast_convert_type(pair, jnp.int32).T,
                      L, axis=1).reshape(-1)
    nrm = jnp.repeat(jnp.sum(emb * emb, axis=1), L)
    wR = jnp.repeat(W_lin[0], L)
    bf = jnp.full((L,), b_lin[0], jnp.float32)
    return _make_sc_call()(xf, embP, nrm, wR, bf)


# speedup vs baseline: 1.1032x; 1.0495x over previous
import functools

import jax
import jax.numpy as jnp
from jax import lax
from jax.experimental import pallas as pl
from jax.experimental.pallas import tpu as pltpu
from jax.experimental.pallas import tpu_sc as plsc

B = 16384
F = 100
D = 16
V = 101
L = 16
XPAD = 128
XSTRIDE = 113
NC = 2
NS = 16
NW = NC * NS
ROWS_PER_W = B // NW
GROUPS = ROWS_PER_W // L


def _fm_body(x_hbm, embP_hbm, nrm_hbm, wR_hbm, b_hbm, out_hbm,
             x_r, embP_v, nrm_v, wR_v, b_v, out_v):
    wid = lax.axis_index("s") * NC + lax.axis_index("c")
    row0 = wid * ROWS_PER_W

    pltpu.sync_copy(x_hbm.at[pl.ds(row0 * XPAD, ROWS_PER_W * XPAD)], x_r)
    pltpu.sync_copy(embP_hbm, embP_v)
    pltpu.sync_copy(nrm_hbm, nrm_v)
    pltpu.sync_copy(wR_hbm, wR_v)
    pltpu.sync_copy(b_hbm, b_v)

    lanes = lax.iota(jnp.int32, L)
    zeros = jnp.zeros((L,), jnp.float32)
    hi_mask = jnp.full((L,), -65536, jnp.int32)

    def g_body(g, carry):
        rows = lanes + g * L
        rbase = rows * XPAD

        def f_body(f, fcarry):
            wf, lin, S, q = fcarry
            xv = plsc.load_gather(x_r, [rbase + wf])
            xc = jnp.clip(xv, 0.0, float(V - 1))
            idx = xc.astype(jnp.int32)
            wv = plsc.load_gather(wR_v, [wf * L + lanes])
            lin = lin + wv * xc
            ea = idx * L + lanes
            nv = plsc.load_gather(nrm_v, [ea])
            q = q + nv
            S_new = []
            for p in range(D // 2):
                w2 = plsc.load_gather(embP_v.at[pl.ds(p * V * L, V * L)], [ea])
                e0 = lax.bitcast_convert_type(
                    jnp.left_shift(w2, 16), jnp.float32)
                e1 = lax.bitcast_convert_type(
                    jnp.bitwise_and(w2, hi_mask), jnp.float32)
                S_new.append(S[2 * p] + e0)
                S_new.append(S[2 * p + 1] + e1)
            wf = wf + 1
            wf = jnp.where(wf >= F, wf - F, wf)
            return wf, lin, tuple(S_new), q

        init = (lanes, b_v[...], (zeros,) * D, zeros)
        _, lin, S, q = lax.fori_loop(0, F, f_body, init, unroll=8)
        sq = zeros
        for d in range(D):
            sq = sq + S[d] * S[d]
        out_v[pl.ds(g * L, L)] = lin + 0.5 * (sq - q)
        return carry

    lax.fori_loop(0, GROUPS, g_body, 0)
    pltpu.sync_copy(out_v, out_hbm.at[pl.ds(row0, ROWS_PER_W)])


def _make_sc_call(interpret=False):
    mesh = plsc.VectorSubcoreMesh(core_axis_name="c", subcore_axis_name="s")
    return pl.kernel(
        _fm_body,
        out_type=jax.ShapeDtypeStruct((B,), jnp.float32),
        mesh=mesh,
        scratch_types=[
            pltpu.VMEM((ROWS_PER_W * XPAD,), jnp.float32),
            pltpu.VMEM((D // 2 * V * L,), jnp.int32),
            pltpu.VMEM((V * L,), jnp.float32),
            pltpu.VMEM((F * L,), jnp.float32),
            pltpu.VMEM((L,), jnp.float32),
            pltpu.VMEM((ROWS_PER_W,), jnp.float32),
        ],
        compiler_params=pltpu.CompilerParams(
            use_tc_tiling_on_sc=False,
            needs_layout_passes=False,
        ),
        interpret=interpret,
    )


@jax.jit
def kernel(x, W_lin, b_lin, emb):
    xf = jnp.pad(x, ((0, 0), (0, XPAD - F))).reshape(-1)
    u = lax.bitcast_convert_type(
        emb.astype(jnp.bfloat16), jnp.uint16).astype(jnp.uint32)
    pair = u[:, 0::2] | (u[:, 1::2] << 16)
    embP = jnp.repeat(lax.bitcast_convert_type(pair, jnp.int32).T,
                      L, axis=1).reshape(-1)
    nrm = jnp.repeat(jnp.sum(emb * emb, axis=1), L)
    wR = jnp.repeat(W_lin[0], L)
    bf = jnp.full((L,), b_lin[0], jnp.float32)
    return _make_sc_call()(xf, embP, nrm, wR, bf)


# speedup vs baseline: 1.1341x; 1.0280x over previous
import functools

import jax
import jax.numpy as jnp
from jax import lax
from jax.experimental import pallas as pl
from jax.experimental.pallas import tpu as pltpu
from jax.experimental.pallas import tpu_sc as plsc

B = 16384
F = 100
D = 16
V = 101
L = 16
XPAD = 128
XSTRIDE = 113
NC = 2
NS = 16
NW = NC * NS
ROWS_PER_W = B // NW
GROUPS = ROWS_PER_W // L


def _fm_body(x_hbm, embP_hbm, nrm_hbm, wR_hbm, b_hbm, out_hbm,
             x_r, embP_v, nrm_v, wR_v, b_v, out_v):
    wid = lax.axis_index("s") * NC + lax.axis_index("c")
    row0 = wid * ROWS_PER_W

    pltpu.sync_copy(x_hbm.at[pl.ds(row0 * XPAD, ROWS_PER_W * XPAD)], x_r)
    pltpu.sync_copy(embP_hbm, embP_v)
    pltpu.sync_copy(nrm_hbm, nrm_v)
    pltpu.sync_copy(wR_hbm, wR_v)
    pltpu.sync_copy(b_hbm, b_v)

    lanes = lax.iota(jnp.int32, L)
    zeros = jnp.zeros((L,), jnp.float32)
    hi_mask = jnp.full((L,), -65536, jnp.int32)

    def g_body(g, carry):
        rows = lanes + g * L
        rbase = rows * XPAD

        def f_body(f, fcarry):
            wf, lin, S, q = fcarry
            xv = plsc.load_gather(x_r, [rbase + wf])
            idx = xv.astype(jnp.int32)
            wv = plsc.load_gather(wR_v, [wf * L + lanes])
            lin = lin + wv * xv
            ea = idx * L + lanes
            nv = plsc.load_gather(nrm_v, [ea])
            q = q + nv
            S_new = []
            for p in range(D // 2):
                w2 = plsc.load_gather(embP_v.at[pl.ds(p * V * L, V * L)], [ea])
                e0 = lax.bitcast_convert_type(
                    jnp.left_shift(w2, 16), jnp.float32)
                e1 = lax.bitcast_convert_type(
                    jnp.bitwise_and(w2, hi_mask), jnp.float32)
                S_new.append(S[2 * p] + e0)
                S_new.append(S[2 * p + 1] + e1)
            wf = wf + 1
            wf = jnp.where(wf >= F, wf - F, wf)
            return wf, lin, tuple(S_new), q

        init = (lanes, b_v[...], (zeros,) * D, zeros)
        _, lin, S, q = lax.fori_loop(0, F, f_body, init, unroll=10)
        sq = zeros
        for d in range(D):
            sq = sq + S[d] * S[d]
        out_v[pl.ds(g * L, L)] = lin + 0.5 * (sq - q)
        return carry

    lax.fori_loop(0, GROUPS, g_body, 0)
    pltpu.sync_copy(out_v, out_hbm.at[pl.ds(row0, ROWS_PER_W)])


def _make_sc_call(interpret=False):
    mesh = plsc.VectorSubcoreMesh(core_axis_name="c", subcore_axis_name="s")
    return pl.kernel(
        _fm_body,
        out_type=jax.ShapeDtypeStruct((B,), jnp.float32),
        mesh=mesh,
        scratch_types=[
            pltpu.VMEM((ROWS_PER_W * XPAD,), jnp.float32),
            pltpu.VMEM((D // 2 * V * L,), jnp.int32),
            pltpu.VMEM((V * L,), jnp.float32),
            pltpu.VMEM((F * L,), jnp.float32),
            pltpu.VMEM((L,), jnp.float32),
            pltpu.VMEM((ROWS_PER_W,), jnp.float32),
        ],
        compiler_params=pltpu.CompilerParams(
            use_tc_tiling_on_sc=False,
            needs_layout_passes=False,
        ),
        interpret=interpret,
    )


@jax.jit
def kernel(x, W_lin, b_lin, emb):
    xf = jnp.pad(x, ((0, 0), (0, XPAD - F))).reshape(-1)
    u = lax.bitcast_convert_type(
        emb.astype(jnp.bfloat16), jnp.uint16).astype(jnp.uint32)
    pair = u[:, 0::2] | (u[:, 1::2] << 16)
    embP = jnp.repeat(lax.bitcast_convert_type(pair, jnp.int32).T,
                      L, axis=1).reshape(-1)
    nrm = jnp.repeat(jnp.sum(emb * emb, axis=1), L)
    wR = jnp.repeat(W_lin[0], L)
    bf = jnp.full((L,), b_lin[0], jnp.float32)
    return _make_sc_call()(xf, embP, nrm, wR, bf)


# speedup vs baseline: 1.1670x; 1.0290x over previous
import functools

import jax
import jax.numpy as jnp
from jax import lax
from jax.experimental import pallas as pl
from jax.experimental.pallas import tpu as pltpu
from jax.experimental.pallas import tpu_sc as plsc

B = 16384
F = 100
D = 16
V = 101
L = 16
XPAD = 128
XSTRIDE = 113
NC = 2
NS = 16
NW = NC * NS
ROWS_PER_W = B // NW
GROUPS = ROWS_PER_W // L


def _fm_body(x_hbm, embP_hbm, nrm_hbm, wR_hbm, b_hbm, out_hbm,
             x_r, embP_v, nrm_v, wR_v, b_v, out_v):
    wid = lax.axis_index("s") * NC + lax.axis_index("c")
    row0 = wid * ROWS_PER_W

    pltpu.sync_copy(x_hbm.at[pl.ds(row0 * XPAD, ROWS_PER_W * XPAD)], x_r)
    pltpu.sync_copy(embP_hbm, embP_v)
    pltpu.sync_copy(nrm_hbm, nrm_v)
    pltpu.sync_copy(wR_hbm, wR_v)
    pltpu.sync_copy(b_hbm, b_v)

    lanes = lax.iota(jnp.int32, L)
    zeros = jnp.zeros((L,), jnp.float32)
    hi_mask = jnp.full((L,), -65536, jnp.int32)

    def g_body(g, carry):
        rows = lanes + g * L
        rbase = rows * XPAD

        def f_body(f, fcarry):
            wf, lin, S, q = fcarry
            xv = plsc.load_gather(x_r, [rbase + wf])
            idx = xv.astype(jnp.int32)
            wv = plsc.load_gather(wR_v, [wf * L + lanes])
            lin = lin + wv * xv
            ea = idx * L + lanes
            nv = plsc.load_gather(nrm_v, [ea])
            q = q + nv
            S_new = []
            for p in range(D // 2):
                w2 = plsc.load_gather(embP_v.at[pl.ds(p * V * L, V * L)], [ea])
                e0 = lax.bitcast_convert_type(
                    jnp.left_shift(w2, 16), jnp.float32)
                e1 = lax.bitcast_convert_type(
                    jnp.bitwise_and(w2, hi_mask), jnp.float32)
                S_new.append(S[2 * p] + e0)
                S_new.append(S[2 * p + 1] + e1)
            wf = wf + 1
            wf = jnp.where(wf >= F, wf - F, wf)
            return wf, lin, tuple(S_new), q

        init = (lanes, b_v[...], (zeros,) * D, zeros)
        _, lin, S, q = lax.fori_loop(0, F, f_body, init, unroll=20)
        sq = zeros
        for d in range(D):
            sq = sq + S[d] * S[d]
        out_v[pl.ds(g * L, L)] = lin + 0.5 * (sq - q)
        return carry

    lax.fori_loop(0, GROUPS, g_body, 0)
    pltpu.sync_copy(out_v, out_hbm.at[pl.ds(row0, ROWS_PER_W)])


def _make_sc_call(interpret=False):
    mesh = plsc.VectorSubcoreMesh(core_axis_name="c", subcore_axis_name="s")
    return pl.kernel(
        _fm_body,
        out_type=jax.ShapeDtypeStruct((B,), jnp.float32),
        mesh=mesh,
        scratch_types=[
            pltpu.VMEM((ROWS_PER_W * XPAD,), jnp.float32),
            pltpu.VMEM((D // 2 * V * L,), jnp.int32),
            pltpu.VMEM((V * L,), jnp.float32),
            pltpu.VMEM((F * L,), jnp.float32),
            pltpu.VMEM((L,), jnp.float32),
            pltpu.VMEM((ROWS_PER_W,), jnp.float32),
        ],
        compiler_params=pltpu.CompilerParams(
            use_tc_tiling_on_sc=False,
            needs_layout_passes=False,
        ),
        interpret=interpret,
    )


@jax.jit
def kernel(x, W_lin, b_lin, emb):
    xf = jnp.pad(x, ((0, 0), (0, XPAD - F))).reshape(-1)
    u = lax.bitcast_convert_type(
        emb.astype(jnp.bfloat16), jnp.uint16).astype(jnp.uint32)
    pair = u[:, 0::2] | (u[:, 1::2] << 16)
    embP = jnp.repeat(lax.bitcast_convert_type(pair, jnp.int32).T,
                      L, axis=1).reshape(-1)
    nrm = jnp.repeat(jnp.sum(emb * emb, axis=1), L)
    wR = jnp.repeat(W_lin[0], L)
    bf = jnp.full((L,), b_lin[0], jnp.float32)
    return _make_sc_call()(xf, embP, nrm, wR, bf)
